# baseline (speedup 1.0000x reference)
import functools

import jax
import jax.numpy as jnp
from jax import lax
from jax.experimental import pallas as pl
from jax.experimental.pallas import tpu as pltpu
from jax.experimental.pallas import tpu_sc as plsc

NC = 2
NS = 16
NW = NC * NS
CHUNK = 80


def _node_proj(x, WT, b):
    n, d = x.shape
    bn = 2000
    grid = n // bn

    def body(x_ref, w_ref, b_ref, o_ref):
        o_ref[...] = jnp.maximum(
            jnp.dot(x_ref[...], w_ref[...], preferred_element_type=jnp.float32)
            + b_ref[...], 0.0)

    return pl.pallas_call(
        body,
        grid=(grid,),
        in_specs=[
            pl.BlockSpec((bn, d), lambda i: (i, 0)),
            pl.BlockSpec(WT.shape, lambda i: (0, 0)),
            pl.BlockSpec(b.shape, lambda i: (0, 0)),
        ],
        out_specs=pl.BlockSpec((bn, WT.shape[1]), lambda i: (i, 0)),
        out_shape=jax.ShapeDtypeStruct((n, WT.shape[1]), jnp.float32),
    )(x, WT, b)


def _film_params(cond, WcT, bc, d, batch2):
    g = cond.shape[0]

    def body(c_ref, w_ref, b_ref, bat_ref, gam_ref, bet_ref, lo_ref, hi_ref):
        gb = jnp.dot(c_ref[...], w_ref[...],
                     preferred_element_type=jnp.float32) + b_ref[...]
        gam_ref[...] = gb[:, :d] + 1.0
        bet_ref[...] = gb[:, d:]
        iota = lax.broadcasted_iota(jnp.int32, (1, g), 1)
        b = bat_ref[...]
        lo_ref[...] = jnp.sum((b < iota).astype(jnp.int32), axis=0,
                              keepdims=True)
        hi_ref[...] = jnp.sum((b < iota + 1).astype(jnp.int32), axis=0,
                              keepdims=True)

    return pl.pallas_call(
        body,
        out_shape=(jax.ShapeDtypeStruct((g, d), jnp.float32),
                   jax.ShapeDtypeStruct((g, d), jnp.float32),
                   jax.ShapeDtypeStruct((1, g), jnp.int32),
                   jax.ShapeDtypeStruct((1, g), jnp.int32)),
    )(cond, WcT, bc, batch2)


def _edge_mlp(jp, spatial, dstc, lo, hi, gamma, beta, WsT, bs,
              Wf1loT, Wf1hiT, Wf2T):
    e, dp = jp.shape
    d = Wf1loT.shape[1]
    g = gamma.shape[0]
    be = 4000
    grid = e // be

    def body(jp_ref, sp_ref, dst_ref, lo_ref, hi_b_ref, gam_ref,
             bet_ref, ws_ref, bs_ref, w1lo_ref, w1hi_ref, w2_ref, o_ref):
        s = jnp.maximum(
            jnp.dot(sp_ref[...], ws_ref[...],
                    preferred_element_type=jnp.float32) + bs_ref[...], 0.0)
        u = lax.bitcast_convert_type(jp_ref[...], jnp.uint32)
        plo = lax.bitcast_convert_type(u << 16, jnp.float32)
        phi = lax.bitcast_convert_type(u & jnp.uint32(0xFFFF0000),
                                       jnp.float32)
        x = (jnp.dot(plo, w1lo_ref[...],
                     preferred_element_type=jnp.float32)
             + jnp.dot(phi, w1hi_ref[...],
                       preferred_element_type=jnp.float32)
             + jnp.dot(s, w2_ref[...], preferred_element_type=jnp.float32))
        mu = jnp.mean(x, axis=-1, keepdims=True)
        dx = x - mu
        var = jnp.mean(dx * dx, axis=-1, keepdims=True)
        xn = dx * lax.rsqrt(var + 1e-5)
        dst = dst_ref[...]
        oh = ((dst >= lo_ref[...]) & (dst < hi_b_ref[...])
              ).astype(jnp.float32)
        ge = jnp.dot(oh, gam_ref[...], preferred_element_type=jnp.float32)
        bee = jnp.dot(oh, bet_ref[...], preferred_element_type=jnp.float32)
        y = jnp.maximum(xn * ge + bee, 0.0)
        uy = lax.bitcast_convert_type(y, jnp.int32)
        r = lax.shift_right_logical(
            uy + 0x7FFF + (lax.shift_right_logical(uy, 16) & 1), 16)
        o_ref[...] = r[:, :d // 2] | lax.shift_left(r[:, d // 2:], 16)

    return pl.pallas_call(
        body,
        grid=(grid,),
        in_specs=[
            pl.BlockSpec((be, dp), lambda i: (i, 0)),
            pl.BlockSpec((be, spatial.shape[1]), lambda i: (i, 0)),
            pl.BlockSpec((be, 1), lambda i: (i, 0)),
            pl.BlockSpec((1, g), lambda i: (0, 0)),
            pl.BlockSpec((1, g), lambda i: (0, 0)),
            pl.BlockSpec(gamma.shape, lambda i: (0, 0)),
            pl.BlockSpec(beta.shape, lambda i: (0, 0)),
            pl.BlockSpec(WsT.shape, lambda i: (0, 0)),
            pl.BlockSpec(bs.shape, lambda i: (0, 0)),
            pl.BlockSpec(Wf1loT.shape, lambda i: (0, 0)),
            pl.BlockSpec(Wf1hiT.shape, lambda i: (0, 0)),
            pl.BlockSpec(Wf2T.shape, lambda i: (0, 0)),
        ],
        out_specs=pl.BlockSpec((be, d // 2), lambda i: (i, 0)),
        out_shape=jax.ShapeDtypeStruct((e, d // 2), jnp.int32),
    )(jp, spatial, dstc, lo, hi, gamma, beta, WsT, bs,
      Wf1loT, Wf1hiT, Wf2T)


def _sc_gather(h, src3, dst3):
    n, d = h.shape
    kw = src3.shape[1]
    e = NW * kw * CHUNK
    mesh = plsc.VectorSubcoreMesh(core_axis_name="c", subcore_axis_name="s")

    assert kw % 2 == 1
    npairs = (kw - 1) // 2
    dh = d // 2

    @functools.partial(
        pl.kernel,
        mesh=mesh,
        out_type=jax.ShapeDtypeStruct((e, dh), jnp.int32),
        scratch_types=[
            pltpu.VMEM((kw, CHUNK), jnp.int32),
            pltpu.VMEM((kw, CHUNK), jnp.int32),
            pltpu.VMEM((2, CHUNK, d), jnp.float32),
            pltpu.VMEM((2, CHUNK, d), jnp.float32),
            pltpu.VMEM((2, CHUNK, dh), jnp.int32),
            pltpu.SemaphoreType.DMA,
            pltpu.SemaphoreType.DMA,
            pltpu.SemaphoreType.DMA,
            pltpu.SemaphoreType.DMA,
        ],
    )
    def k(h_hbm, src_hbm, dst_hbm, joint_hbm,
          src_v, dst_v, rows_i, rows_j, prod,
          sem_a, sem_b, st_a, st_b):
        wid = lax.axis_index("s") * NC + lax.axis_index("c")
        row0 = wid * kw
        pltpu.sync_copy(src_hbm.at[wid], src_v)
        pltpu.sync_copy(dst_hbm.at[wid], dst_v)

        sems = (sem_a, sem_b)
        st_sems = (st_a, st_b)

        def issue(kk, buf):
            pltpu.async_copy(h_hbm.at[src_v.at[kk]], rows_i.at[buf],
                             sems[buf])
            pltpu.async_copy(h_hbm.at[dst_v.at[kk]], rows_j.at[buf],
                             sems[buf])

        def drain(kk, buf):
            pltpu.make_async_copy(h_hbm.at[src_v.at[kk]], rows_i.at[buf],
                                  sems[buf]).wait()
            pltpu.make_async_copy(h_hbm.at[dst_v.at[kk]], rows_j.at[buf],
                                  sems[buf]).wait()

        def out_slice(kk):
            return joint_hbm.at[pl.ds((row0 + kk) * CHUNK, CHUNK), :]

        def rne(f):
            u = lax.bitcast_convert_type(f, jnp.int32)
            return lax.shift_right_logical(
                u + 0x7FFF + (lax.shift_right_logical(u, 16) & 1), 16)

        def process(kk, buf, first):
            drain(kk, buf)
            if not first:
                pltpu.make_async_copy(prod.at[buf], out_slice(kk),
                                      st_sems[buf]).wait()

            def mul_row(r, c2):
                for c in range(dh // 16):
                    sa = pl.ds(c * 16, 16)
                    sb = pl.ds(dh + c * 16, 16)
                    pa = rows_i[buf, r, sa] * rows_j[buf, r, sa]
                    pb = rows_i[buf, r, sb] * rows_j[buf, r, sb]
                    prod[buf, r, sa] = rne(pa) | lax.shift_left(rne(pb), 16)
                return c2

            lax.fori_loop(0, CHUNK, mul_row, 0)
            pltpu.async_copy(prod.at[buf], out_slice(kk), st_sems[buf])

        issue(0, 0)
        issue(1, 1)
        process(0, 0, True)
        issue(2, 0)
        process(1, 1, True)
        issue(3, 1)

        def body(t, carry):
            k0 = 2 * t + 2
            process(k0, 0, False)
            issue(k0 + 2, 0)
            process(k0 + 1, 1, False)
            issue(k0 + 3, 1)
            return carry

        lax.fori_loop(0, npairs - 2, body, 0)
        process(kw - 3, 0, False)
        issue(kw - 1, 0)
        process(kw - 2, 1, False)
        process(kw - 1, 0, False)
        pltpu.make_async_copy(prod.at[0], out_slice(kw - 1), st_a).wait()
        pltpu.make_async_copy(prod.at[1], out_slice(kw - 2), st_b).wait()

    return k(h, src3, dst3)


def _sc_scatter(ef, dst4, n, d):
    dh = ef.shape[1]
    nph = dst4.shape[1]
    kwp = dst4.shape[2]
    chs = dst4.shape[3]
    kw = nph * kwp
    assert ef.shape[0] == NW * kw * chs and kwp % 2 == 1 and kwp >= 5
    nzc = n // chs
    zper = -(-nzc // NS)
    mesh = plsc.VectorSubcoreMesh(core_axis_name="c", subcore_axis_name="s")

    @functools.partial(
        pl.kernel,
        mesh=mesh,
        out_type=jax.ShapeDtypeStruct((NC, n, d), jnp.float32),
        scratch_types=[
            pltpu.VMEM_SHARED((n, d), jnp.float32),
            pltpu.VMEM((kwp, chs), jnp.int32),
            pltpu.VMEM((2, chs, dh), jnp.int32),
            pltpu.VMEM((2, chs, d), jnp.float32),
            pltpu.SemaphoreType.DMA,
            pltpu.SemaphoreType.DMA,
            pltpu.SemaphoreType.DMA,
            pltpu.SemaphoreType.DMA,
        ],
    )
    def k(ef_hbm, dst_hbm, out_hbm, acc, dst_v, rows_v, rows_f,
          sem_a, sem_b, sc_a, sc_b):
        cid = lax.axis_index("c")
        sid = lax.axis_index("s")
        wid = sid * NC + cid
        row0 = wid * kw
        sems = (sem_a, sem_b)
        sc_sems = (sc_a, sc_b)

        def zrow(r, c2):
            for c8 in range(d // 16):
                rows_f[0, r, pl.ds(c8 * 16, 16)] = jnp.zeros((16,),
                                                             jnp.float32)
            return c2

        lax.fori_loop(0, chs, zrow, 0)
        for j in range(zper):
            ch = sid + j * NS
            @pl.when(ch < nzc)
            def _():
                pltpu.sync_copy(rows_f.at[0],
                                acc.at[pl.ds(ch * chs, chs), :])
        plsc.subcore_barrier()

        for ph in range(nph):
            base = ph * kwp
            pltpu.sync_copy(dst_hbm.at[wid, ph], dst_v)

            def ef_slice(kk, base=base):
                return ef_hbm.at[pl.ds((row0 + base + kk) * chs, chs), :]

            def issue(kk, buf, ef_slice=ef_slice):
                pltpu.async_copy(ef_slice(kk), rows_v.at[buf], sems[buf])

            def drain(kk, buf, ef_slice=ef_slice):
                pltpu.make_async_copy(ef_slice(kk), rows_v.at[buf],
                                      sems[buf]).wait()

            def scat_copy(kk, buf):
                return pltpu.make_async_copy(rows_f.at[buf],
                                             acc.at[dst_v.at[kk]],
                                             sc_sems[buf])

            def process(kk, buf, first, drain=drain, scat_copy=scat_copy):
                drain(kk, buf)
                if not first:
                    scat_copy(kk, buf).wait()

                def unp_row(r, c2):
                    for c in range(dh // 16):
                        w = rows_v[buf, r, pl.ds(c * 16, 16)]
                        rows_f[buf, r, pl.ds(c * 16, 16)] = (
                            lax.bitcast_convert_type(lax.shift_left(w, 16),
                                                     jnp.float32))
                        rows_f[buf, r, pl.ds(dh + c * 16, 16)] = (
                            lax.bitcast_convert_type(w & jnp.int32(-65536),
                                                     jnp.float32))
                    return c2

                lax.fori_loop(0, chs, unp_row, 0)
                pltpu.async_copy(rows_f.at[buf], acc.at[dst_v.at[kk]],
                                 sc_sems[buf], add=True)

            issue(0, 0)
            issue(1, 1)
            process(0, 0, True)
            issue(2, 0)
            process(1, 1, True)
            issue(3, 1)

            def body(t, carry, process=process, issue=issue):
                k0 = 2 * t + 2
                process(k0, 0, False)
                issue(k0 + 2, 0)
                process(k0 + 1, 1, False)
                issue(k0 + 3, 1)
                return carry

            lax.fori_loop(0, (kwp - 1) // 2 - 2, body, 0)
            process(kwp - 3, 0, False)
            issue(kwp - 1, 0)
            process(kwp - 2, 1, False)
            process(kwp - 1, 0, False)
            scat_copy(kwp - 1, 0).wait()
            scat_copy(kwp - 2, 1).wait()
        plsc.subcore_barrier()

        for j in range(zper):
            ch = sid + j * NS
            @pl.when(ch < nzc)
            def _():
                pltpu.sync_copy(acc.at[pl.ds(ch * chs, chs), :],
                                rows_f.at[0])
                pltpu.sync_copy(rows_f.at[0],
                                out_hbm.at[cid, pl.ds(ch * chs, chs), :])

    return k(ef, dst4)


def kernel(node_feats, cond_feats, edge_index, batch_ids, spatial_feats,
           W_node, b_node, W_s, b_s, W_cond, b_cond, W_film):
    n, dd = node_feats.shape
    e = edge_index.shape[1]

    h = _node_proj(node_feats, W_node.T, b_node.reshape(1, -1))
    gamma, beta, lo, hi = _film_params(cond_feats, W_cond.T,
                                       b_cond.reshape(1, -1), dd,
                                       batch_ids.reshape(n, 1))

    kw = e // (NW * CHUNK)
    src3 = edge_index[0].reshape(NW, kw, CHUNK)
    dst3 = edge_index[1].reshape(NW, kw, CHUNK)
    jp = _sc_gather(h, src3, dst3)
    wf1 = W_film[:, :dd].T
    ef = _edge_mlp(jp, spatial_feats, edge_index[1].reshape(e, 1),
                   lo, hi, gamma, beta, W_s.T, b_s.reshape(1, -1),
                   wf1[:dd // 2], wf1[dd // 2:], W_film[:, dd:].T)

    chs = 40
    dst4 = edge_index[1].reshape(NW, 2, e // (NW * chs * 2), chs)
    parts = _sc_scatter(ef, dst4, n, dd)
    return parts[0] + parts[1]


# speedup vs baseline: 1.2609x; 1.2609x over previous
import functools

import jax
import jax.numpy as jnp
from jax import lax
from jax.experimental import pallas as pl
from jax.experimental.pallas import tpu as pltpu
from jax.experimental.pallas import tpu_sc as plsc

NC = 2
NS = 16
NW = NC * NS
CHUNK = 80


def _node_proj(x, WT, b):
    n, d = x.shape
    bn = 2000
    grid = n // bn

    def body(x_ref, w_ref, b_ref, o_ref):
        o_ref[...] = jnp.maximum(
            jnp.dot(x_ref[...], w_ref[...], preferred_element_type=jnp.float32)
            + b_ref[...], 0.0)

    return pl.pallas_call(
        body,
        grid=(grid,),
        in_specs=[
            pl.BlockSpec((bn, d), lambda i: (i, 0)),
            pl.BlockSpec(WT.shape, lambda i: (0, 0)),
            pl.BlockSpec(b.shape, lambda i: (0, 0)),
        ],
        out_specs=pl.BlockSpec((bn, WT.shape[1]), lambda i: (i, 0)),
        out_shape=jax.ShapeDtypeStruct((n, WT.shape[1]), jnp.float32),
    )(x, WT, b)


def _film_params(cond, WcT, bc, d, batch2):
    g = cond.shape[0]

    def body(c_ref, w_ref, b_ref, bat_ref, gam_ref, bet_ref, lo_ref, hi_ref):
        gb = jnp.dot(c_ref[...], w_ref[...],
                     preferred_element_type=jnp.float32) + b_ref[...]
        gam_ref[...] = gb[:, :d] + 1.0
        bet_ref[...] = gb[:, d:]
        iota = lax.broadcasted_iota(jnp.int32, (1, g), 1)
        b = bat_ref[...]
        lo_ref[...] = jnp.sum((b < iota).astype(jnp.int32), axis=0,
                              keepdims=True)
        hi_ref[...] = jnp.sum((b < iota + 1).astype(jnp.int32), axis=0,
                              keepdims=True)

    return pl.pallas_call(
        body,
        out_shape=(jax.ShapeDtypeStruct((g, d), jnp.float32),
                   jax.ShapeDtypeStruct((g, d), jnp.float32),
                   jax.ShapeDtypeStruct((1, g), jnp.int32),
                   jax.ShapeDtypeStruct((1, g), jnp.int32)),
    )(cond, WcT, bc, batch2)


def _edge_mlp(jp, spatial, dstc, lo, hi, gamma, beta, WsT, bs,
              Wf1loT, Wf1hiT, Wf2T):
    e, dp = jp.shape
    d = Wf1loT.shape[1]
    g = gamma.shape[0]
    be = 4000
    grid = e // be

    def body(jp_ref, sp_ref, dst_ref, lo_ref, hi_b_ref, gam_ref,
             bet_ref, ws_ref, bs_ref, w1lo_ref, w1hi_ref, w2_ref, o_ref):
        s = jnp.maximum(
            jnp.dot(sp_ref[...], ws_ref[...],
                    preferred_element_type=jnp.float32) + bs_ref[...], 0.0)
        u = lax.bitcast_convert_type(jp_ref[...], jnp.uint32)
        plo = lax.bitcast_convert_type(u << 16, jnp.float32)
        phi = lax.bitcast_convert_type(u & jnp.uint32(0xFFFF0000),
                                       jnp.float32)
        x = (jnp.dot(plo, w1lo_ref[...],
                     preferred_element_type=jnp.float32)
             + jnp.dot(phi, w1hi_ref[...],
                       preferred_element_type=jnp.float32)
             + jnp.dot(s, w2_ref[...], preferred_element_type=jnp.float32))
        mu = jnp.mean(x, axis=-1, keepdims=True)
        dx = x - mu
        var = jnp.mean(dx * dx, axis=-1, keepdims=True)
        xn = dx * lax.rsqrt(var + 1e-5)
        dst = dst_ref[...]
        oh = ((dst >= lo_ref[...]) & (dst < hi_b_ref[...])
              ).astype(jnp.float32)
        ge = jnp.dot(oh, gam_ref[...], preferred_element_type=jnp.float32)
        bee = jnp.dot(oh, bet_ref[...], preferred_element_type=jnp.float32)
        o_ref[...] = jnp.maximum(xn * ge + bee, 0.0)

    return pl.pallas_call(
        body,
        grid=(grid,),
        in_specs=[
            pl.BlockSpec((be, dp), lambda i: (i, 0)),
            pl.BlockSpec((be, spatial.shape[1]), lambda i: (i, 0)),
            pl.BlockSpec((be, 1), lambda i: (i, 0)),
            pl.BlockSpec((1, g), lambda i: (0, 0)),
            pl.BlockSpec((1, g), lambda i: (0, 0)),
            pl.BlockSpec(gamma.shape, lambda i: (0, 0)),
            pl.BlockSpec(beta.shape, lambda i: (0, 0)),
            pl.BlockSpec(WsT.shape, lambda i: (0, 0)),
            pl.BlockSpec(bs.shape, lambda i: (0, 0)),
            pl.BlockSpec(Wf1loT.shape, lambda i: (0, 0)),
            pl.BlockSpec(Wf1hiT.shape, lambda i: (0, 0)),
            pl.BlockSpec(Wf2T.shape, lambda i: (0, 0)),
        ],
        out_specs=pl.BlockSpec((be, d), lambda i: (i, 0)),
        out_shape=jax.ShapeDtypeStruct((e, d), jnp.float32),
    )(jp, spatial, dstc, lo, hi, gamma, beta, WsT, bs,
      Wf1loT, Wf1hiT, Wf2T)


def _sc_gather(h, src3, dst3):
    n, d = h.shape
    kw = src3.shape[1]
    e = NW * kw * CHUNK
    mesh = plsc.VectorSubcoreMesh(core_axis_name="c", subcore_axis_name="s")

    assert kw % 2 == 1
    npairs = (kw - 1) // 2
    dh = d // 2

    @functools.partial(
        pl.kernel,
        mesh=mesh,
        out_type=jax.ShapeDtypeStruct((e, dh), jnp.int32),
        scratch_types=[
            pltpu.VMEM((kw, CHUNK), jnp.int32),
            pltpu.VMEM((kw, CHUNK), jnp.int32),
            pltpu.VMEM((2, CHUNK, d), jnp.float32),
            pltpu.VMEM((2, CHUNK, d), jnp.float32),
            pltpu.VMEM((2, CHUNK, dh), jnp.int32),
            pltpu.SemaphoreType.DMA,
            pltpu.SemaphoreType.DMA,
            pltpu.SemaphoreType.DMA,
            pltpu.SemaphoreType.DMA,
        ],
    )
    def k(h_hbm, src_hbm, dst_hbm, joint_hbm,
          src_v, dst_v, rows_i, rows_j, prod,
          sem_a, sem_b, st_a, st_b):
        wid = lax.axis_index("s") * NC + lax.axis_index("c")
        row0 = wid * kw
        pltpu.sync_copy(src_hbm.at[wid], src_v)
        pltpu.sync_copy(dst_hbm.at[wid], dst_v)

        sems = (sem_a, sem_b)
        st_sems = (st_a, st_b)

        def issue(kk, buf):
            pltpu.async_copy(h_hbm.at[src_v.at[kk]], rows_i.at[buf],
                             sems[buf])
            pltpu.async_copy(h_hbm.at[dst_v.at[kk]], rows_j.at[buf],
                             sems[buf])

        def drain(kk, buf):
            pltpu.make_async_copy(h_hbm.at[src_v.at[kk]], rows_i.at[buf],
                                  sems[buf]).wait()
            pltpu.make_async_copy(h_hbm.at[dst_v.at[kk]], rows_j.at[buf],
                                  sems[buf]).wait()

        def out_slice(kk):
            return joint_hbm.at[pl.ds((row0 + kk) * CHUNK, CHUNK), :]

        def rne(f):
            u = lax.bitcast_convert_type(f, jnp.int32)
            return lax.shift_right_logical(
                u + 0x7FFF + (lax.shift_right_logical(u, 16) & 1), 16)

        def process(kk, buf, first):
            drain(kk, buf)
            if not first:
                pltpu.make_async_copy(prod.at[buf], out_slice(kk),
                                      st_sems[buf]).wait()

            def mul_row(r, c2):
                for c in range(dh // 16):
                    sa = pl.ds(c * 16, 16)
                    sb = pl.ds(dh + c * 16, 16)
                    pa = rows_i[buf, r, sa] * rows_j[buf, r, sa]
                    pb = rows_i[buf, r, sb] * rows_j[buf, r, sb]
                    prod[buf, r, sa] = rne(pa) | lax.shift_left(rne(pb), 16)
                return c2

            lax.fori_loop(0, CHUNK, mul_row, 0)
            pltpu.async_copy(prod.at[buf], out_slice(kk), st_sems[buf])

        issue(0, 0)
        issue(1, 1)
        process(0, 0, True)
        issue(2, 0)
        process(1, 1, True)
        issue(3, 1)

        def body(t, carry):
            k0 = 2 * t + 2
            process(k0, 0, False)
            issue(k0 + 2, 0)
            process(k0 + 1, 1, False)
            issue(k0 + 3, 1)
            return carry

        lax.fori_loop(0, npairs - 2, body, 0)
        process(kw - 3, 0, False)
        issue(kw - 1, 0)
        process(kw - 2, 1, False)
        process(kw - 1, 0, False)
        pltpu.make_async_copy(prod.at[0], out_slice(kw - 1), st_a).wait()
        pltpu.make_async_copy(prod.at[1], out_slice(kw - 2), st_b).wait()

    return k(h, src3, dst3)


def _sc_scatter(ef, dst3, n, d):
    kw = dst3.shape[1]
    assert ef.shape[0] == NW * kw * CHUNK and kw % 2 == 1
    npairs = (kw - 1) // 2
    nzc = n // CHUNK
    zper = -(-nzc // NS)
    mesh = plsc.VectorSubcoreMesh(core_axis_name="c", subcore_axis_name="s")

    @functools.partial(
        pl.kernel,
        mesh=mesh,
        out_type=jax.ShapeDtypeStruct((NC, n, d), jnp.float32),
        scratch_types=[
            pltpu.VMEM_SHARED((n, d), jnp.float32),
            pltpu.VMEM((kw, CHUNK), jnp.int32),
            pltpu.VMEM((2, CHUNK, d), jnp.float32),
            pltpu.SemaphoreType.DMA,
            pltpu.SemaphoreType.DMA,
        ],
    )
    def k(ef_hbm, dst_hbm, out_hbm, acc, dst_v, rows_v, sem_a, sem_b):
        cid = lax.axis_index("c")
        sid = lax.axis_index("s")
        wid = sid * NC + cid
        row0 = wid * kw
        sems = (sem_a, sem_b)

        def zrow(r, c2):
            for c8 in range(d // 16):
                rows_v[0, r, pl.ds(c8 * 16, 16)] = jnp.zeros((16,),
                                                             jnp.float32)
            return c2

        lax.fori_loop(0, CHUNK, zrow, 0)
        for j in range(zper):
            ch = sid + j * NS
            @pl.when(ch < nzc)
            def _():
                pltpu.sync_copy(rows_v.at[0],
                                acc.at[pl.ds(ch * CHUNK, CHUNK), :])
        plsc.subcore_barrier()

        pltpu.sync_copy(dst_hbm.at[wid], dst_v)

        def ef_slice(kk):
            return ef_hbm.at[pl.ds((row0 + kk) * CHUNK, CHUNK), :]

        def issue(kk, buf):
            pltpu.async_copy(ef_slice(kk), rows_v.at[buf], sems[buf])

        def drain(kk, buf):
            pltpu.make_async_copy(ef_slice(kk), rows_v.at[buf],
                                  sems[buf]).wait()

        def scat(kk, buf):
            pltpu.sync_copy(rows_v.at[buf], acc.at[dst_v.at[kk]], add=True)

        issue(0, 0)

        def body(t, carry):
            k0 = 2 * t
            issue(k0 + 1, 1)
            drain(k0, 0)
            scat(k0, 0)
            issue(k0 + 2, 0)
            drain(k0 + 1, 1)
            scat(k0 + 1, 1)
            return carry

        lax.fori_loop(0, npairs, body, 0)
        drain(kw - 1, 0)
        scat(kw - 1, 0)
        plsc.subcore_barrier()

        for j in range(zper):
            ch = sid + j * NS
            @pl.when(ch < nzc)
            def _():
                pltpu.sync_copy(acc.at[pl.ds(ch * CHUNK, CHUNK), :],
                                rows_v.at[0])
                pltpu.sync_copy(rows_v.at[0],
                                out_hbm.at[cid, pl.ds(ch * CHUNK, CHUNK), :])

    return k(ef, dst3)


def kernel(node_feats, cond_feats, edge_index, batch_ids, spatial_feats,
           W_node, b_node, W_s, b_s, W_cond, b_cond, W_film):
    n, dd = node_feats.shape
    e = edge_index.shape[1]

    h = _node_proj(node_feats, W_node.T, b_node.reshape(1, -1))
    gamma, beta, lo, hi = _film_params(cond_feats, W_cond.T,
                                       b_cond.reshape(1, -1), dd,
                                       batch_ids.reshape(n, 1))

    kw = e // (NW * CHUNK)
    src3 = edge_index[0].reshape(NW, kw, CHUNK)
    dst3 = edge_index[1].reshape(NW, kw, CHUNK)
    jp = _sc_gather(h, src3, dst3)
    wf1 = W_film[:, :dd].T
    ef = _edge_mlp(jp, spatial_feats, edge_index[1].reshape(e, 1),
                   lo, hi, gamma, beta, W_s.T, b_s.reshape(1, -1),
                   wf1[:dd // 2], wf1[dd // 2:], W_film[:, dd:].T)

    parts = _sc_scatter(ef, dst3, n, dd)
    return parts[0] + parts[1]
